# baseline (speedup 1.0000x reference)
import functools

import jax
import jax.numpy as jnp
from jax.experimental import pallas as pl
from jax.experimental.pallas import tpu as pltpu

_N_FRAGS = 16384
_IN_DIM = 4096
_N_EXPERTS = 64
_TOP_K = 8
_CHUNK = 512
_NCHUNKS = _N_FRAGS // _CHUNK
_NBUF = 6
_NOBUF = 4


def _route_chunk(logits):
    work = logits
    sel = jnp.zeros(logits.shape, dtype=jnp.bool_)
    row_max = None
    for t in range(_TOP_K):
        m = jnp.max(work, axis=-1, keepdims=True)
        if t == 0:
            row_max = m
        hit = work == m
        sel = jnp.logical_or(sel, hit)
        work = jnp.where(hit, -jnp.inf, work)
    e = jnp.where(sel, jnp.exp(logits - row_max), 0.0)
    return e / jnp.sum(e, axis=-1, keepdims=True)


def _router_body(x_hbm, wt_ref, w_hbm, load_ref, xbuf, wbuf, sem_in, sem_out):
    def fetch(c):
        pltpu.make_async_copy(
            x_hbm.at[pl.ds(c * _CHUNK, _CHUNK), :],
            xbuf.at[c % _NBUF],
            sem_in.at[c % _NBUF],
        ).start()

    for c in range(_NBUF):
        fetch(c)

    wtb = wt_ref[...].astype(jnp.bfloat16)
    acc = jnp.zeros((1, _N_EXPERTS), dtype=jnp.float32)

    for c in range(_NCHUNKS):
        slot = c % _NBUF
        oslot = c % _NOBUF
        pltpu.make_async_copy(
            x_hbm.at[pl.ds(c * _CHUNK, _CHUNK), :],
            xbuf.at[slot],
            sem_in.at[slot],
        ).wait()

        logits = jnp.dot(
            xbuf[slot].astype(jnp.bfloat16), wtb,
            preferred_element_type=jnp.float32,
        )
        weights = _route_chunk(logits)
        acc = acc + jnp.sum(weights, axis=0, keepdims=True)

        if c >= _NOBUF:
            pltpu.make_async_copy(
                wbuf.at[oslot],
                w_hbm.at[pl.ds((c - _NOBUF) * _CHUNK, _CHUNK), :],
                sem_out.at[oslot],
            ).wait()
        wbuf[oslot] = weights
        pltpu.make_async_copy(
            wbuf.at[oslot],
            w_hbm.at[pl.ds(c * _CHUNK, _CHUNK), :],
            sem_out.at[oslot],
        ).start()

        if c + _NBUF < _NCHUNKS:
            fetch(c + _NBUF)

    for c in range(_NCHUNKS - _NOBUF, _NCHUNKS):
        oslot = c % _NOBUF
        pltpu.make_async_copy(
            wbuf.at[oslot],
            w_hbm.at[pl.ds(c * _CHUNK, _CHUNK), :],
            sem_out.at[oslot],
        ).wait()

    load_ref[...] = acc * (1.0 / _N_FRAGS)


@functools.partial(jax.jit)
def kernel(x, W):
    wt = W.T
    weights, load = pl.pallas_call(
        _router_body,
        in_specs=[
            pl.BlockSpec(memory_space=pl.ANY),
            pl.BlockSpec((_IN_DIM, _N_EXPERTS), lambda: (0, 0)),
        ],
        out_specs=[
            pl.BlockSpec(memory_space=pl.ANY),
            pl.BlockSpec((1, _N_EXPERTS), lambda: (0, 0)),
        ],
        out_shape=[
            jax.ShapeDtypeStruct((_N_FRAGS, _N_EXPERTS), jnp.float32),
            jax.ShapeDtypeStruct((1, _N_EXPERTS), jnp.float32),
        ],
        scratch_shapes=[
            pltpu.VMEM((_NBUF, _CHUNK, _IN_DIM), jnp.float32),
            pltpu.VMEM((_NOBUF, _CHUNK, _N_EXPERTS), jnp.float32),
            pltpu.SemaphoreType.DMA((_NBUF,)),
            pltpu.SemaphoreType.DMA((_NOBUF,)),
        ],
    )(x, wt)
    return weights, load.reshape(_N_EXPERTS)


# speedup vs baseline: 1.0989x; 1.0989x over previous
import functools

import jax
import jax.numpy as jnp
from jax.experimental import pallas as pl
from jax.experimental.pallas import tpu as pltpu

_N_FRAGS = 16384
_IN_DIM = 4096
_N_EXPERTS = 64
_TOP_K = 8
_BLOCK_ROWS = 512
_HALF = _N_FRAGS // 2
_GRID = _HALF // _BLOCK_ROWS


def _route_chunk(logits):
    work = logits
    sel = jnp.zeros(logits.shape, dtype=jnp.bool_)
    row_max = None
    for t in range(_TOP_K):
        m = jnp.max(work, axis=-1, keepdims=True)
        if t == 0:
            row_max = m
        hit = work == m
        sel = jnp.logical_or(sel, hit)
        work = jnp.where(hit, -jnp.inf, work)
    e = jnp.where(sel, jnp.exp(logits - row_max), 0.0)
    return e / jnp.sum(e, axis=-1, keepdims=True)


def _router_block(xa_ref, xb_ref, wt_ref, wa_ref, wb_ref, part_ref):
    wtb = wt_ref[...].astype(jnp.bfloat16)
    la = jnp.dot(xa_ref[...].astype(jnp.bfloat16), wtb,
                 preferred_element_type=jnp.float32)
    lb = jnp.dot(xb_ref[...].astype(jnp.bfloat16), wtb,
                 preferred_element_type=jnp.float32)
    wa = _route_chunk(la)
    wb = _route_chunk(lb)
    wa_ref[...] = wa
    wb_ref[...] = wb
    part = jnp.sum(wa, axis=0, keepdims=True) + jnp.sum(wb, axis=0, keepdims=True)
    part_ref[...] = part[None] * (1.0 / _N_FRAGS)


@functools.partial(jax.jit)
def kernel(x, W):
    wt = W.T
    wa, wb, parts = pl.pallas_call(
        _router_block,
        grid=(_GRID,),
        in_specs=[
            pl.BlockSpec((_BLOCK_ROWS, _IN_DIM), lambda i: (i, 0)),
            pl.BlockSpec((_BLOCK_ROWS, _IN_DIM), lambda i: (i + _GRID, 0)),
            pl.BlockSpec((_IN_DIM, _N_EXPERTS), lambda i: (0, 0)),
        ],
        out_specs=[
            pl.BlockSpec((_BLOCK_ROWS, _N_EXPERTS), lambda i: (i, 0)),
            pl.BlockSpec((_BLOCK_ROWS, _N_EXPERTS), lambda i: (i, 0)),
            pl.BlockSpec((1, 1, _N_EXPERTS), lambda i: (i, 0, 0)),
        ],
        out_shape=[
            jax.ShapeDtypeStruct((_HALF, _N_EXPERTS), jnp.float32),
            jax.ShapeDtypeStruct((_HALF, _N_EXPERTS), jnp.float32),
            jax.ShapeDtypeStruct((_GRID, 1, _N_EXPERTS), jnp.float32),
        ],
    )(x, x, wt)
    weights = jnp.concatenate([wa, wb], axis=0)
    return weights, parts.sum(axis=(0, 1))


# speedup vs baseline: 1.1917x; 1.0844x over previous
import functools

import jax
import jax.numpy as jnp
from jax.experimental import pallas as pl
from jax.experimental.pallas import tpu as pltpu

_N_FRAGS = 16384
_IN_DIM = 4096
_N_EXPERTS = 64
_TOP_K = 8
_BLOCK_ROWS = 1024
_GRID = _N_FRAGS // _BLOCK_ROWS


def _router_block(x_ref, wt_ref, w_out_ref, part_ref):
    logits = jnp.dot(
        x_ref[...].astype(jnp.bfloat16),
        wt_ref[...].astype(jnp.bfloat16),
        preferred_element_type=jnp.float32,
    )

    work = logits
    sel = jnp.zeros(logits.shape, dtype=jnp.bool_)
    row_max = None
    for t in range(_TOP_K):
        m = jnp.max(work, axis=-1, keepdims=True)
        if t == 0:
            row_max = m
        hit = work == m
        sel = jnp.logical_or(sel, hit)
        work = jnp.where(hit, -jnp.inf, work)

    e = jnp.where(sel, jnp.exp(logits - row_max), 0.0)
    weights = e / jnp.sum(e, axis=-1, keepdims=True)
    w_out_ref[...] = weights
    part_ref[...] = jnp.sum(weights, axis=0, keepdims=True)[None] * (1.0 / _N_FRAGS)


@functools.partial(jax.jit)
def kernel(x, W):
    wt = W.T
    weights, parts = pl.pallas_call(
        _router_block,
        grid=(_GRID,),
        in_specs=[
            pl.BlockSpec((_BLOCK_ROWS, _IN_DIM), lambda i: (i, 0)),
            pl.BlockSpec((_IN_DIM, _N_EXPERTS), lambda i: (0, 0)),
        ],
        out_specs=[
            pl.BlockSpec((_BLOCK_ROWS, _N_EXPERTS), lambda i: (i, 0)),
            pl.BlockSpec((1, 1, _N_EXPERTS), lambda i: (i, 0, 0)),
        ],
        out_shape=[
            jax.ShapeDtypeStruct((_N_FRAGS, _N_EXPERTS), jnp.float32),
            jax.ShapeDtypeStruct((_GRID, 1, _N_EXPERTS), jnp.float32),
        ],
        compiler_params=pltpu.CompilerParams(
            dimension_semantics=("parallel",),
        ),
    )(x, wt)
    return weights, parts.sum(axis=(0, 1))
